# baseline (speedup 1.0000x reference)
import functools

import jax
import jax.numpy as jnp
from jax import lax
from jax.experimental import pallas as pl
from jax.experimental.pallas import tpu as pltpu
from jax.experimental.pallas import tpu_sc as plsc

EMB = 256
NBINS = 8192
NTOK = 8 * 576
COMMIT_COST = 0.25

TM = 512
TN = 2048


def _argmin_body(a2_ref, w_ref, a2n_ref, b2_ref, idx_ref, minv_ref):
    j = pl.program_id(1)
    a2x = a2_ref[...]
    w = w_ref[...]
    s2 = lax.dot_general(a2x, w, (((1,), (1,)), ((), ())),
                         preferred_element_type=jnp.float32)
    d2 = (a2n_ref[...] + b2_ref[...]) - s2
    v = jnp.sqrt(jnp.maximum(d2, 0.0))
    u = jnp.min(v, axis=1, keepdims=True)
    col = lax.broadcasted_iota(jnp.int32, (TM, TN), 1)
    targ = jnp.min(jnp.where(v == u, col, jnp.int32(2**30)),
                   axis=1, keepdims=True) + j * TN

    @pl.when(j == 0)
    def _init():
        minv_ref[...] = u
        idx_ref[...] = targ

    @pl.when(j > 0)
    def _update():
        old = minv_ref[...]
        better = u < old
        minv_ref[...] = jnp.where(better, u, old)
        idx_ref[...] = jnp.where(better, targ, idx_ref[...])


def _nearest_codes(flat_x, W, a2, b2):
    grid = (NTOK // TM, NBINS // TN)
    return pl.pallas_call(
        _argmin_body,
        grid=grid,
        in_specs=[
            pl.BlockSpec((TM, EMB), lambda t, j: (t, 0)),
            pl.BlockSpec((TN, EMB), lambda t, j: (j, 0)),
            pl.BlockSpec((TM, 1), lambda t, j: (t, 0)),
            pl.BlockSpec((1, TN), lambda t, j: (0, j)),
        ],
        out_specs=[
            pl.BlockSpec((TM, 1), lambda t, j: (t, 0)),
            pl.BlockSpec((TM, 1), lambda t, j: (t, 0)),
        ],
        out_shape=[
            jax.ShapeDtypeStruct((NTOK, 1), jnp.int32),
            jax.ShapeDtypeStruct((NTOK, 1), jnp.float32),
        ],
    )(flat_x, W, a2, b2)


def _sc_gather(W, idx):
    info = plsc.get_sparse_core_info()
    nw = info.num_cores * info.num_subcores
    bpw = NTOK // nw
    nch = 2
    ch = bpw // nch
    mesh = plsc.VectorSubcoreMesh(core_axis_name="c", subcore_axis_name="s")

    @functools.partial(
        pl.kernel, mesh=mesh,
        out_type=jax.ShapeDtypeStruct((NTOK, EMB), jnp.float32),
        scratch_types=[
            pltpu.VMEM((nch, ch), jnp.int32),
            pltpu.VMEM((bpw, EMB), jnp.float32),
            pltpu.SemaphoreType.DMA,
        ],
    )
    def k(w_hbm, idx_hbm, out_hbm, idx_v, rows_v, sem):
        wid = lax.axis_index("s") * info.num_cores + lax.axis_index("c")
        base = wid * bpw
        for c in range(nch):
            pltpu.sync_copy(idx_hbm.at[pl.ds(base + c * ch, ch)], idx_v.at[c])
        copies = [
            pltpu.async_copy(w_hbm.at[idx_v.at[c]],
                             rows_v.at[pl.ds(c * ch, ch)], sem)
            for c in range(nch)
        ]
        for cp in copies:
            cp.wait()
        pltpu.sync_copy(rows_v, out_hbm.at[pl.ds(base, bpw)])

    return k(W, idx)


def _scalars_body(g_ref, x_ref, minv_ref, loss_ref, fit_ref):
    d = g_ref[...] - x_ref[...]
    ssq = jnp.sum(d * d)
    loss_ref[0, 0] = (1.0 + COMMIT_COST) * ssq / (NTOK * EMB)
    fit_ref[0, 0] = jnp.sum(minv_ref[...]) / NTOK


def _scalars(G, x_raw, minv):
    return pl.pallas_call(
        _scalars_body,
        out_specs=[
            pl.BlockSpec(memory_space=pltpu.SMEM),
            pl.BlockSpec(memory_space=pltpu.SMEM),
        ],
        out_shape=[
            jax.ShapeDtypeStruct((1, 1), jnp.float32),
            jax.ShapeDtypeStruct((1, 1), jnp.float32),
        ],
    )(G, x_raw, minv)


def kernel(x, W):
    N, width, T = x.shape
    flat_x = jnp.transpose(x, (0, 2, 1)).reshape(-1, width)
    a2 = jnp.sum(flat_x * flat_x, axis=1)[:, None]
    b2 = jnp.sum(W * W, axis=1)[None, :]
    idx, minv = _nearest_codes(flat_x + flat_x, W, a2, b2)
    idx = idx.reshape(-1)
    minv = minv.reshape(-1)
    G = _sc_gather(W, idx)
    x_raw = x.reshape(NTOK, EMB)
    loss, fit = _scalars(G, x_raw, minv)
    quantized_out = G.reshape(N, width, T)
    return (quantized_out, loss.reshape(()), fit.reshape(()))


# speedup vs baseline: 1.0501x; 1.0501x over previous
import functools

import jax
import jax.numpy as jnp
from jax import lax
from jax.experimental import pallas as pl
from jax.experimental.pallas import tpu as pltpu
from jax.experimental.pallas import tpu_sc as plsc

EMB = 256
NBINS = 8192
NTOK = 8 * 576
COMMIT_COST = 0.25

TM = 512
TN = 2048


def _argmin_body(a2_ref, w_ref, a2n_ref, b2_ref, idx_ref, minv_ref):
    j = pl.program_id(1)
    a2x = a2_ref[...]
    w = w_ref[...]
    s2 = lax.dot_general(a2x, w, (((1,), (1,)), ((), ())),
                         preferred_element_type=jnp.float32)
    d2 = (a2n_ref[...] + b2_ref[...]) - s2
    v = jnp.sqrt(jnp.maximum(d2, 0.0))
    u = jnp.min(v, axis=1, keepdims=True)
    colf = lax.broadcasted_iota(jnp.int32, (TM, TN), 1).astype(jnp.float32)
    targf = jnp.min(jnp.where(v == u, colf, jnp.float32(2.0**30)),
                    axis=1, keepdims=True)
    targ = targf.astype(jnp.int32) + j * TN

    @pl.when(j == 0)
    def _init():
        minv_ref[...] = u
        idx_ref[...] = targ

    @pl.when(j > 0)
    def _update():
        old = minv_ref[...]
        better = u < old
        minv_ref[...] = jnp.where(better, u, old)
        idx_ref[...] = jnp.where(better, targ, idx_ref[...])


def _nearest_codes(flat_x, W, a2, b2):
    grid = (NTOK // TM, NBINS // TN)
    return pl.pallas_call(
        _argmin_body,
        grid=grid,
        in_specs=[
            pl.BlockSpec((TM, EMB), lambda t, j: (t, 0)),
            pl.BlockSpec((TN, EMB), lambda t, j: (j, 0)),
            pl.BlockSpec((TM, 1), lambda t, j: (t, 0)),
            pl.BlockSpec((1, TN), lambda t, j: (0, j)),
        ],
        out_specs=[
            pl.BlockSpec((TM, 1), lambda t, j: (t, 0)),
            pl.BlockSpec((TM, 1), lambda t, j: (t, 0)),
        ],
        out_shape=[
            jax.ShapeDtypeStruct((NTOK, 1), jnp.int32),
            jax.ShapeDtypeStruct((NTOK, 1), jnp.float32),
        ],
    )(flat_x, W, a2, b2)


def _sc_gather(W, idx):
    info = plsc.get_sparse_core_info()
    nw = info.num_cores * info.num_subcores
    bpw = NTOK // nw
    nch = 2
    ch = bpw // nch
    mesh = plsc.VectorSubcoreMesh(core_axis_name="c", subcore_axis_name="s")

    @functools.partial(
        pl.kernel, mesh=mesh,
        out_type=jax.ShapeDtypeStruct((NTOK, EMB), jnp.float32),
        scratch_types=[
            pltpu.VMEM((nch, ch), jnp.int32),
            pltpu.VMEM((bpw, EMB), jnp.float32),
            pltpu.SemaphoreType.DMA,
        ],
    )
    def k(w_hbm, idx_hbm, out_hbm, idx_v, rows_v, sem):
        wid = lax.axis_index("s") * info.num_cores + lax.axis_index("c")
        base = wid * bpw
        for c in range(nch):
            pltpu.sync_copy(idx_hbm.at[pl.ds(base + c * ch, ch)], idx_v.at[c])
        copies = [
            pltpu.async_copy(w_hbm.at[idx_v.at[c]],
                             rows_v.at[pl.ds(c * ch, ch)], sem)
            for c in range(nch)
        ]
        for cp in copies:
            cp.wait()
        pltpu.sync_copy(rows_v, out_hbm.at[pl.ds(base, bpw)])

    return k(W, idx)


def _scalars_body(g_ref, x_ref, minv_ref, loss_ref, fit_ref):
    d = g_ref[...] - x_ref[...]
    ssq = jnp.sum(d * d)
    loss_ref[0, 0] = (1.0 + COMMIT_COST) * ssq / (NTOK * EMB)
    fit_ref[0, 0] = jnp.sum(minv_ref[...]) / NTOK


def _scalars(G, x_raw, minv):
    return pl.pallas_call(
        _scalars_body,
        out_specs=[
            pl.BlockSpec(memory_space=pltpu.SMEM),
            pl.BlockSpec(memory_space=pltpu.SMEM),
        ],
        out_shape=[
            jax.ShapeDtypeStruct((1, 1), jnp.float32),
            jax.ShapeDtypeStruct((1, 1), jnp.float32),
        ],
    )(G, x_raw, minv)


def kernel(x, W):
    N, width, T = x.shape
    flat_x = jnp.transpose(x, (0, 2, 1)).reshape(-1, width)
    a2 = jnp.sum(flat_x * flat_x, axis=1)[:, None]
    b2 = jnp.sum(W * W, axis=1)[None, :]
    idx, minv = _nearest_codes(flat_x + flat_x, W, a2, b2)
    idx = idx.reshape(-1)
    minv = minv.reshape(-1)
    G = _sc_gather(W, idx)
    x_raw = x.reshape(NTOK, EMB)
    loss, fit = _scalars(G, x_raw, minv)
    quantized_out = G.reshape(N, width, T)
    return (quantized_out, loss.reshape(()), fit.reshape(()))


# speedup vs baseline: 1.0773x; 1.0260x over previous
import functools

import jax
import jax.numpy as jnp
from jax import lax
from jax.experimental import pallas as pl
from jax.experimental.pallas import tpu as pltpu
from jax.experimental.pallas import tpu_sc as plsc

EMB = 256
NBINS = 8192
NTOK = 8 * 576
COMMIT_COST = 0.25

TM = 512
TN = 4096


def _argmin_body(a2_ref, w_ref, a2n_ref, b2_ref, idx_ref, minv_ref):
    j = pl.program_id(1)
    a2x = a2_ref[...]
    w = w_ref[...]
    s2 = lax.dot_general(a2x, w, (((1,), (1,)), ((), ())),
                         preferred_element_type=jnp.float32)
    d2 = (a2n_ref[...] + b2_ref[...]) - s2
    v = jnp.sqrt(jnp.maximum(d2, 0.0))
    u = jnp.min(v, axis=1, keepdims=True)
    colf = lax.broadcasted_iota(jnp.int32, (TM, TN), 1).astype(jnp.float32)
    targf = jnp.min(jnp.where(v == u, colf, jnp.float32(2.0**30)),
                    axis=1, keepdims=True)
    targ = targf.astype(jnp.int32) + j * TN

    @pl.when(j == 0)
    def _init():
        minv_ref[...] = u
        idx_ref[...] = targ

    @pl.when(j > 0)
    def _update():
        old = minv_ref[...]
        better = u < old
        minv_ref[...] = jnp.where(better, u, old)
        idx_ref[...] = jnp.where(better, targ, idx_ref[...])


def _nearest_codes(flat_x, W, a2, b2):
    grid = (NTOK // TM, NBINS // TN)
    return pl.pallas_call(
        _argmin_body,
        grid=grid,
        in_specs=[
            pl.BlockSpec((TM, EMB), lambda t, j: (t, 0)),
            pl.BlockSpec((TN, EMB), lambda t, j: (j, 0)),
            pl.BlockSpec((TM, 1), lambda t, j: (t, 0)),
            pl.BlockSpec((1, TN), lambda t, j: (0, j)),
        ],
        out_specs=[
            pl.BlockSpec((TM, 1), lambda t, j: (t, 0)),
            pl.BlockSpec((TM, 1), lambda t, j: (t, 0)),
        ],
        out_shape=[
            jax.ShapeDtypeStruct((NTOK, 1), jnp.int32),
            jax.ShapeDtypeStruct((NTOK, 1), jnp.float32),
        ],
    )(flat_x, W, a2, b2)


def _sc_gather(W, idx):
    info = plsc.get_sparse_core_info()
    nw = info.num_cores * info.num_subcores
    bpw = NTOK // nw
    nch = 2
    ch = bpw // nch
    mesh = plsc.VectorSubcoreMesh(core_axis_name="c", subcore_axis_name="s")

    @functools.partial(
        pl.kernel, mesh=mesh,
        out_type=jax.ShapeDtypeStruct((NTOK, EMB), jnp.float32),
        scratch_types=[
            pltpu.VMEM((nch, ch), jnp.int32),
            pltpu.VMEM((bpw, EMB), jnp.float32),
            pltpu.SemaphoreType.DMA,
        ],
    )
    def k(w_hbm, idx_hbm, out_hbm, idx_v, rows_v, sem):
        wid = lax.axis_index("s") * info.num_cores + lax.axis_index("c")
        base = wid * bpw
        for c in range(nch):
            pltpu.sync_copy(idx_hbm.at[pl.ds(base + c * ch, ch)], idx_v.at[c])
        copies = [
            pltpu.async_copy(w_hbm.at[idx_v.at[c]],
                             rows_v.at[pl.ds(c * ch, ch)], sem)
            for c in range(nch)
        ]
        for cp in copies:
            cp.wait()
        pltpu.sync_copy(rows_v, out_hbm.at[pl.ds(base, bpw)])

    return k(W, idx)


def _scalars_body(g_ref, x_ref, minv_ref, loss_ref, fit_ref):
    d = g_ref[...] - x_ref[...]
    ssq = jnp.sum(d * d)
    loss_ref[0, 0] = (1.0 + COMMIT_COST) * ssq / (NTOK * EMB)
    fit_ref[0, 0] = jnp.sum(minv_ref[...]) / NTOK


def _scalars(G, x_raw, minv):
    return pl.pallas_call(
        _scalars_body,
        out_specs=[
            pl.BlockSpec(memory_space=pltpu.SMEM),
            pl.BlockSpec(memory_space=pltpu.SMEM),
        ],
        out_shape=[
            jax.ShapeDtypeStruct((1, 1), jnp.float32),
            jax.ShapeDtypeStruct((1, 1), jnp.float32),
        ],
    )(G, x_raw, minv)


def kernel(x, W):
    N, width, T = x.shape
    flat_x = jnp.transpose(x, (0, 2, 1)).reshape(-1, width)
    a2 = jnp.sum(flat_x * flat_x, axis=1)[:, None]
    b2 = jnp.sum(W * W, axis=1)[None, :]
    idx, minv = _nearest_codes(flat_x + flat_x, W, a2, b2)
    idx = idx.reshape(-1)
    minv = minv.reshape(-1)
    G = _sc_gather(W, idx)
    x_raw = x.reshape(NTOK, EMB)
    loss, fit = _scalars(G, x_raw, minv)
    quantized_out = G.reshape(N, width, T)
    return (quantized_out, loss.reshape(()), fit.reshape(()))


# speedup vs baseline: 1.0986x; 1.0197x over previous
import functools

import jax
import jax.numpy as jnp
from jax import lax
from jax.experimental import pallas as pl
from jax.experimental.pallas import tpu as pltpu
from jax.experimental.pallas import tpu_sc as plsc

EMB = 256
NBINS = 8192
NTOK = 8 * 576
COMMIT_COST = 0.25

TM = 512
TN = 8192


def _argmin_body(a2_ref, w_ref, a2n_ref, b2_ref, idx_ref, minv_ref):
    j = pl.program_id(1)
    a2x = a2_ref[...]
    w = w_ref[...]
    s2 = lax.dot_general(a2x, w, (((1,), (1,)), ((), ())),
                         preferred_element_type=jnp.float32)
    d2 = (a2n_ref[...] + b2_ref[...]) - s2
    v = jnp.sqrt(jnp.maximum(d2, 0.0))
    u = jnp.min(v, axis=1, keepdims=True)
    colf = lax.broadcasted_iota(jnp.int32, (TM, TN), 1).astype(jnp.float32)
    targf = jnp.min(jnp.where(v == u, colf, jnp.float32(2.0**30)),
                    axis=1, keepdims=True)
    targ = targf.astype(jnp.int32) + j * TN

    @pl.when(j == 0)
    def _init():
        minv_ref[...] = u
        idx_ref[...] = targ

    @pl.when(j > 0)
    def _update():
        old = minv_ref[...]
        better = u < old
        minv_ref[...] = jnp.where(better, u, old)
        idx_ref[...] = jnp.where(better, targ, idx_ref[...])


def _nearest_codes(flat_x, W, a2, b2):
    grid = (NTOK // TM, NBINS // TN)
    return pl.pallas_call(
        _argmin_body,
        grid=grid,
        in_specs=[
            pl.BlockSpec((TM, EMB), lambda t, j: (t, 0)),
            pl.BlockSpec((TN, EMB), lambda t, j: (j, 0)),
            pl.BlockSpec((TM, 1), lambda t, j: (t, 0)),
            pl.BlockSpec((1, TN), lambda t, j: (0, j)),
        ],
        out_specs=[
            pl.BlockSpec((TM, 1), lambda t, j: (t, 0)),
            pl.BlockSpec((TM, 1), lambda t, j: (t, 0)),
        ],
        out_shape=[
            jax.ShapeDtypeStruct((NTOK, 1), jnp.int32),
            jax.ShapeDtypeStruct((NTOK, 1), jnp.float32),
        ],
    )(flat_x, W, a2, b2)


def _sc_gather(W, idx):
    info = plsc.get_sparse_core_info()
    nw = info.num_cores * info.num_subcores
    bpw = NTOK // nw
    nch = 2
    ch = bpw // nch
    mesh = plsc.VectorSubcoreMesh(core_axis_name="c", subcore_axis_name="s")

    @functools.partial(
        pl.kernel, mesh=mesh,
        out_type=jax.ShapeDtypeStruct((NTOK, EMB), jnp.float32),
        scratch_types=[
            pltpu.VMEM((nch, ch), jnp.int32),
            pltpu.VMEM((bpw, EMB), jnp.float32),
            pltpu.SemaphoreType.DMA,
        ],
    )
    def k(w_hbm, idx_hbm, out_hbm, idx_v, rows_v, sem):
        wid = lax.axis_index("s") * info.num_cores + lax.axis_index("c")
        base = wid * bpw
        for c in range(nch):
            pltpu.sync_copy(idx_hbm.at[pl.ds(base + c * ch, ch)], idx_v.at[c])
        copies = [
            pltpu.async_copy(w_hbm.at[idx_v.at[c]],
                             rows_v.at[pl.ds(c * ch, ch)], sem)
            for c in range(nch)
        ]
        for cp in copies:
            cp.wait()
        pltpu.sync_copy(rows_v, out_hbm.at[pl.ds(base, bpw)])

    return k(W, idx)


def _scalars_body(g_ref, x_ref, minv_ref, loss_ref, fit_ref):
    d = g_ref[...] - x_ref[...]
    ssq = jnp.sum(d * d)
    loss_ref[0, 0] = (1.0 + COMMIT_COST) * ssq / (NTOK * EMB)
    fit_ref[0, 0] = jnp.sum(minv_ref[...]) / NTOK


def _scalars(G, x_raw, minv):
    return pl.pallas_call(
        _scalars_body,
        out_specs=[
            pl.BlockSpec(memory_space=pltpu.SMEM),
            pl.BlockSpec(memory_space=pltpu.SMEM),
        ],
        out_shape=[
            jax.ShapeDtypeStruct((1, 1), jnp.float32),
            jax.ShapeDtypeStruct((1, 1), jnp.float32),
        ],
    )(G, x_raw, minv)


def kernel(x, W):
    N, width, T = x.shape
    flat_x = jnp.transpose(x, (0, 2, 1)).reshape(-1, width)
    a2 = jnp.sum(flat_x * flat_x, axis=1)[:, None]
    b2 = jnp.sum(W * W, axis=1)[None, :]
    idx, minv = _nearest_codes(flat_x + flat_x, W, a2, b2)
    idx = idx.reshape(-1)
    minv = minv.reshape(-1)
    G = _sc_gather(W, idx)
    x_raw = x.reshape(NTOK, EMB)
    loss, fit = _scalars(G, x_raw, minv)
    quantized_out = G.reshape(N, width, T)
    return (quantized_out, loss.reshape(()), fit.reshape(()))
